# baseline (speedup 1.0000x reference)
import functools

import jax
import jax.numpy as jnp
from jax import lax
from jax.experimental import pallas as pl
from jax.experimental.pallas import tpu as pltpu, tpu_sc as plsc

_NUM_EMB = 100000
_T = 8
_D = 32
_B = 4096

_NC = 2
_NS = 16
_NW = _NC * _NS
_B_PER_W = _B // _NW

_ROW = _T * _D

_MESH = plsc.VectorSubcoreMesh(core_axis_name="c", subcore_axis_name="s")


@functools.partial(
    pl.kernel,
    mesh=_MESH,
    out_type=jax.ShapeDtypeStruct((_B, _ROW), jnp.float32),
    scratch_types=[
        pltpu.VMEM((_B_PER_W,), jnp.int32),
        pltpu.VMEM((_B_PER_W, _ROW), jnp.float32),
        pltpu.SemaphoreType.DMA,
    ],
)
def _gather_kernel(idx_hbm, table_hbm, out_hbm, idx_v, rows_v, sem):
    wid = lax.axis_index("s") * _NC + lax.axis_index("c")
    base = wid * _B_PER_W
    pltpu.sync_copy(idx_hbm.at[pl.ds(base, _B_PER_W)], idx_v)
    pltpu.async_copy(table_hbm.at[idx_v], rows_v, sem).wait()
    pltpu.sync_copy(rows_v, out_hbm.at[pl.ds(base, _B_PER_W)])


_TB = 512
_TGRID = -(-_NUM_EMB // _TB)


def _tr_body(eye_ref, in_ref, out_ref):
    x = jnp.reshape(in_ref[...], (_ROW, _TB))
    out_ref[...] = lax.dot_general(
        x, eye_ref[...], (((0,), (0,)), ((), ())),
        preferred_element_type=jnp.float32,
    )


_transpose = pl.pallas_call(
    _tr_body,
    grid=(_TGRID,),
    in_specs=[
        pl.BlockSpec((_ROW, _ROW), lambda i: (0, 0)),
        pl.BlockSpec((_T, _D, _TB), lambda i: (0, 0, i)),
    ],
    out_specs=pl.BlockSpec((_TB, _ROW), lambda i: (i, 0)),
    out_shape=jax.ShapeDtypeStruct((_NUM_EMB, _ROW), jnp.float32),
)


def kernel(indices, weight):
    wt = jnp.transpose(weight, (1, 2, 0))
    eye = jnp.eye(_ROW, dtype=jnp.float32)
    table2d = _transpose(eye, wt)
    out = _gather_kernel(indices.astype(jnp.int32), table2d)
    return jnp.reshape(out, (_B, _T, _D))


# speedup vs baseline: 2.0261x; 2.0261x over previous
import functools

import jax
import jax.numpy as jnp
from jax import lax
from jax.experimental import pallas as pl
from jax.experimental.pallas import tpu as pltpu, tpu_sc as plsc

_NUM_EMB = 100000
_T = 8
_D = 32
_B = 4096

_NC = 2
_NS = 16
_NW = _NC * _NS
_B_PER_W = _B // _NW

_ROW = _T * _D

_MESH = plsc.VectorSubcoreMesh(core_axis_name="c", subcore_axis_name="s")


@functools.partial(
    pl.kernel,
    mesh=_MESH,
    out_type=jax.ShapeDtypeStruct((_B, _ROW), jnp.float32),
    scratch_types=[
        pltpu.VMEM((_B_PER_W,), jnp.int32),
        pltpu.VMEM((_B_PER_W, _ROW), jnp.float32),
        pltpu.SemaphoreType.DMA,
    ],
)
def _gather_kernel(idx_hbm, table_hbm, out_hbm, idx_v, rows_v, sem):
    wid = lax.axis_index("s") * _NC + lax.axis_index("c")
    base = wid * _B_PER_W
    pltpu.sync_copy(idx_hbm.at[pl.ds(base, _B_PER_W)], idx_v)
    pltpu.async_copy(table_hbm.at[idx_v], rows_v, sem).wait()
    pltpu.sync_copy(rows_v, out_hbm.at[pl.ds(base, _B_PER_W)])


_TB = 4096
_TGRID = -(-_NUM_EMB // _TB)


def _tr_body(in_ref, out_ref):
    x = jnp.reshape(in_ref[...], (_ROW, _TB))
    out_ref[...] = jnp.transpose(x, (1, 0))


_transpose = pl.pallas_call(
    _tr_body,
    grid=(_TGRID,),
    in_specs=[pl.BlockSpec((_T, _D, _TB), lambda i: (0, 0, i))],
    out_specs=pl.BlockSpec((_TB, _ROW), lambda i: (i, 0)),
    out_shape=jax.ShapeDtypeStruct((_NUM_EMB, _ROW), jnp.float32),
)


def kernel(indices, weight):
    wt = jnp.transpose(weight, (1, 2, 0))
    table2d = _transpose(wt)
    out = _gather_kernel(indices.astype(jnp.int32), table2d)
    return jnp.reshape(out, (_B, _T, _D))


# speedup vs baseline: 2.0775x; 1.0254x over previous
import functools

import jax
import jax.numpy as jnp
from jax import lax
from jax.experimental import pallas as pl
from jax.experimental.pallas import tpu as pltpu, tpu_sc as plsc

_NUM_EMB = 100000
_T = 8
_D = 32
_B = 4096

_NC = 2
_NS = 16
_NW = _NC * _NS
_B_PER_W = _B // _NW

_ROW = _T * _D

_MESH = plsc.VectorSubcoreMesh(core_axis_name="c", subcore_axis_name="s")


@functools.partial(
    pl.kernel,
    mesh=_MESH,
    out_type=jax.ShapeDtypeStruct((_B, _ROW), jnp.float32),
    scratch_types=[
        pltpu.VMEM((_B_PER_W,), jnp.int32),
        pltpu.VMEM((_B_PER_W, _ROW), jnp.float32),
        pltpu.SemaphoreType.DMA,
    ],
)
def _gather_kernel(idx_hbm, table_hbm, out_hbm, idx_v, rows_v, sem):
    wid = lax.axis_index("s") * _NC + lax.axis_index("c")
    base = wid * _B_PER_W
    pltpu.sync_copy(idx_hbm.at[pl.ds(base, _B_PER_W)], idx_v)
    pltpu.async_copy(table_hbm.at[idx_v], rows_v, sem).wait()
    pltpu.sync_copy(rows_v, out_hbm.at[pl.ds(base, _B_PER_W)])


_TB = 8192
_TGRID = -(-_NUM_EMB // _TB)


def _tr_body(in_ref, out_ref):
    x = jnp.reshape(in_ref[...], (_ROW, _TB))
    out_ref[...] = jnp.transpose(x, (1, 0))


_transpose = pl.pallas_call(
    _tr_body,
    grid=(_TGRID,),
    in_specs=[pl.BlockSpec((_T, _D, _TB), lambda i: (0, 0, i))],
    out_specs=pl.BlockSpec((_TB, _ROW), lambda i: (i, 0)),
    out_shape=jax.ShapeDtypeStruct((_NUM_EMB, _ROW), jnp.float32),
)


def kernel(indices, weight):
    wt = jnp.transpose(weight, (1, 2, 0))
    table2d = _transpose(wt)
    out = _gather_kernel(indices.astype(jnp.int32), table2d)
    return jnp.reshape(out, (_B, _T, _D))
